# baseline (speedup 1.0000x reference)
import numpy as np

import jax
import jax.numpy as jnp
from jax import lax
from jax.experimental import pallas as pl
from jax.experimental.pallas import tpu as pltpu
from jax.experimental.pallas import tpu_sc as plsc

N = 10000
E = 320000
D = 128
DH = 64
DW = 8
NC = 2
NS = 16
EW = E // NS
K = 80
NCHUNK = EW // K
NBG = 2
NBS = 3
NBW = 4
RSUB = 624
RTAIL = N - NS * RSUB
R = 1000

_g = np.arange(D) // 32 * 32
_t = np.arange(D) % 32
PERM = (_g + np.where(_t % 2 == 0, _t // 2, 16 + _t // 2)).astype(np.int32)


def _emit_prep(h, hpads_ref, as_ref, ad_ref, sw_ref, atts_ref, attd_ref):
    a_s = jnp.dot(h, atts_ref[...], preferred_element_type=jnp.float32)
    a_d = jnp.dot(h, attd_ref[...], preferred_element_type=jnp.float32)
    hpads_ref[0] = h[:, :DH].astype(jnp.bfloat16)
    hpads_ref[1] = h[:, DH:].astype(jnp.bfloat16)
    as_ref[...] = a_s
    ad_ref[...] = a_d
    s = a_s + a_d
    sw_ref[...] = jnp.exp(jnp.maximum(s, 0.2 * s))


def _combine(acc_ref, den_ref, sw_ref, hpads_ref, pmat_ref, b_ref):
    ht = jnp.concatenate([hpads_ref[0], hpads_ref[1]],
                         axis=1).astype(jnp.float32)
    h = jnp.dot(ht, pmat_ref[...], preferred_element_type=jnp.float32)
    sw = sw_ref[...]
    num = jnp.concatenate([acc_ref[0], acc_ref[1]], axis=1) + sw * h
    den = den_ref[0, :, 0:1] + sw
    return num / den + b_ref[...]


def _prep_body(z_ref, w_ref, atts_ref, attd_ref, hpads_ref, as_ref, ad_ref,
               sw_ref):
    h = jnp.dot(z_ref[...], w_ref[...], preferred_element_type=jnp.float32)
    _emit_prep(h, hpads_ref, as_ref, ad_ref, sw_ref, atts_ref, attd_ref)


def _mid_body(acc_ref, den_ref, sw_ref, hpads_ref, pmat_ref, b_ref, w_ref,
              atts_ref, attd_ref, hpads2_ref, as_ref, ad_ref, sw2_ref):
    z = jnp.maximum(_combine(acc_ref, den_ref, sw_ref, hpads_ref, pmat_ref,
                             b_ref), 0.0)
    h2 = jnp.dot(z, w_ref[...], preferred_element_type=jnp.float32)
    _emit_prep(h2, hpads2_ref, as_ref, ad_ref, sw2_ref, atts_ref, attd_ref)


def _fin_body(acc_ref, den_ref, sw_ref, hpads_ref, pmat_ref, b_ref, out_ref):
    out_ref[...] = _combine(acc_ref, den_ref, sw_ref, hpads_ref, pmat_ref,
                            b_ref)


_prep_outs = dict(
    out_specs=[
        pl.BlockSpec((NC, R, DH), lambda i: (0, i, 0)),
        pl.BlockSpec((R, 1), lambda i: (i, 0)),
        pl.BlockSpec((R, 1), lambda i: (i, 0)),
        pl.BlockSpec((R, 1), lambda i: (i, 0)),
    ],
    out_shape=[
        jax.ShapeDtypeStruct((NC, N, DH), jnp.bfloat16),
        jax.ShapeDtypeStruct((N, 1), jnp.float32),
        jax.ShapeDtypeStruct((N, 1), jnp.float32),
        jax.ShapeDtypeStruct((N, 1), jnp.float32),
    ],
)

_acc_specs = [
    pl.BlockSpec((NC, R, DH), lambda i: (0, i, 0)),
    pl.BlockSpec((NC, R, DW), lambda i: (0, i, 0)),
    pl.BlockSpec((R, 1), lambda i: (i, 0)),
    pl.BlockSpec((NC, R, DH), lambda i: (0, i, 0)),
    pl.BlockSpec((D, D), lambda i: (0, 0)),
    pl.BlockSpec((1, D), lambda i: (0, 0)),
]


def _tc_prep(z, W, att_s, att_d):
    return pl.pallas_call(
        _prep_body,
        grid=(N // R,),
        in_specs=[
            pl.BlockSpec((R, D), lambda i: (i, 0)),
            pl.BlockSpec((D, D), lambda i: (0, 0)),
            pl.BlockSpec((D, 1), lambda i: (0, 0)),
            pl.BlockSpec((D, 1), lambda i: (0, 0)),
        ],
        **_prep_outs,
    )(z, W, att_s, att_d)


def _tc_mid(acc, den, sw, hpads, pmat, b, W, att_s, att_d):
    return pl.pallas_call(
        _mid_body,
        grid=(N // R,),
        in_specs=_acc_specs + [
            pl.BlockSpec((D, D), lambda i: (0, 0)),
            pl.BlockSpec((D, 1), lambda i: (0, 0)),
            pl.BlockSpec((D, 1), lambda i: (0, 0)),
        ],
        **_prep_outs,
    )(acc, den, sw, hpads, pmat, b, W, att_s, att_d)


def _tc_fin(acc, den, sw, hpads, pmat, b):
    return pl.pallas_call(
        _fin_body,
        grid=(N // R,),
        in_specs=_acc_specs,
        out_specs=pl.BlockSpec((R, D), lambda i: (i, 0)),
        out_shape=jax.ShapeDtypeStruct((N, D), jnp.float32),
    )(acc, den, sw, hpads, pmat, b)


def _sc_body(hpads_hbm, as_hbm, ad_hbm, src_hbm, dst_hbm, zeros_hbm, zd_hbm,
             acc_out, den_out,
             srcv, dstv, astab, adtab, wbufs, rows_bf, rows_f, acc_sp, den_sp,
             gsem, ssem, wsem, psem):
    c = lax.axis_index("c")
    s = lax.axis_index("s")

    pltpu.async_copy(src_hbm.at[s], srcv, psem)
    pltpu.async_copy(dst_hbm.at[s], dstv, psem)
    pltpu.async_copy(as_hbm, astab, psem)
    pltpu.async_copy(ad_hbm, adtab, psem)
    pltpu.async_copy(zeros_hbm, acc_sp.at[pl.ds(s * RSUB, RSUB)], psem)
    pltpu.async_copy(zd_hbm.at[pl.ds(0, RSUB)],
                     den_sp.at[pl.ds(s * RSUB, RSUB)], psem)
    pltpu.async_copy(zd_hbm.at[pl.ds(0, NBW * K)], wbufs, psem)

    @pl.when(s == NS - 1)
    def _():
        pltpu.async_copy(zeros_hbm.at[pl.ds(0, RTAIL)],
                         acc_sp.at[pl.ds(NS * RSUB, RTAIL)], psem)
        pltpu.async_copy(zd_hbm.at[pl.ds(0, RTAIL)],
                         den_sp.at[pl.ds(NS * RSUB, RTAIL)], psem)

    pltpu.make_async_copy(src_hbm.at[s], srcv, psem).wait()
    pltpu.make_async_copy(dst_hbm.at[s], dstv, psem).wait()
    pltpu.make_async_copy(as_hbm, astab, psem).wait()
    pltpu.make_async_copy(ad_hbm, adtab, psem).wait()
    pltpu.make_async_copy(zeros_hbm, acc_sp.at[pl.ds(s * RSUB, RSUB)],
                          psem).wait()
    pltpu.make_async_copy(zd_hbm.at[pl.ds(0, RSUB)],
                          den_sp.at[pl.ds(s * RSUB, RSUB)], psem).wait()
    pltpu.make_async_copy(zd_hbm.at[pl.ds(0, NBW * K)], wbufs, psem).wait()

    @pl.when(s == NS - 1)
    def _():
        pltpu.make_async_copy(zeros_hbm.at[pl.ds(0, RTAIL)],
                              acc_sp.at[pl.ds(NS * RSUB, RTAIL)], psem).wait()
        pltpu.make_async_copy(zd_hbm.at[pl.ds(0, RTAIL)],
                              den_sp.at[pl.ds(NS * RSUB, RTAIL)], psem).wait()

    plsc.subcore_barrier()
    htab = hpads_hbm.at[c]
    col0 = jnp.zeros((16,), jnp.int32)
    lane = lax.iota(jnp.int32, 16)

    pltpu.async_copy(htab.at[srcv.at[0]], rows_bf.at[0], gsem)

    def chunk(i, carry):
        bg = lax.rem(i, NBG)
        bs = lax.rem(i, NBS)
        bw = lax.rem(i, NBW)

        @pl.when(i >= NBW)
        def _():
            pltpu.make_async_copy(
                wbufs.at[pl.ds(bw * K, K)],
                den_sp.at[dstv.at[i - NBW]], wsem).wait()

        for j in range(K // 16):
            sidx = srcv[i, pl.ds(j * 16, 16)]
            didx = dstv[i, pl.ds(j * 16, 16)]
            a = plsc.load_gather(astab, [sidx]) + plsc.load_gather(adtab, [didx])
            w16 = jnp.exp(jnp.maximum(a, 0.2 * a))
            plsc.store_scatter(wbufs, [bw * K + j * 16 + lane, col0], w16)
        pltpu.async_copy(wbufs.at[pl.ds(bw * K, K)],
                         den_sp.at[dstv.at[i]], wsem, add=True)

        @pl.when(i < NCHUNK - 1)
        def _():
            pltpu.async_copy(htab.at[srcv.at[i + 1]], rows_bf.at[1 - bg], gsem)

        @pl.when(i >= NBS)
        def _():
            pltpu.make_async_copy(rows_f.at[bs],
                                  acc_sp.at[dstv.at[i - NBS]], ssem).wait()

        pltpu.make_async_copy(htab.at[srcv.at[i]], rows_bf.at[bg], gsem).wait()

        @plsc.parallel_loop(0, K, unroll=16)
        def scale(k):
            wv = plsc.load_gather(wbufs, [jnp.full((16,), bw * K + k,
                                                   jnp.int32), col0])
            for g in range(DH // 32):
                v = rows_bf[bg, k, pl.ds(g * 32, 32)]
                ev, od = plsc.unpack(v, format=plsc.PackFormat.INTERLEAVED,
                                     preferred_element_type=jnp.float32)
                rows_f[bs, k, pl.ds(g * 32, 16)] = ev * wv
                rows_f[bs, k, pl.ds(g * 32 + 16, 16)] = od * wv

        pltpu.async_copy(rows_f.at[bs], acc_sp.at[dstv.at[i]], ssem, add=True)
        return carry

    lax.fori_loop(0, NCHUNK, chunk, 0)
    for t in range(NBS):
        i = NCHUNK - NBS + t
        pltpu.make_async_copy(rows_f.at[lax.rem(i, NBS)],
                              acc_sp.at[dstv.at[i]], ssem).wait()
    for t in range(NBW):
        i = NCHUNK - NBW + t
        pltpu.make_async_copy(wbufs.at[pl.ds(lax.rem(i, NBW) * K, K)],
                              den_sp.at[dstv.at[i]], wsem).wait()
    plsc.subcore_barrier()
    pltpu.sync_copy(acc_sp.at[pl.ds(s * RSUB, RSUB)],
                    acc_out.at[c, pl.ds(s * RSUB, RSUB)])
    pltpu.sync_copy(den_sp.at[pl.ds(s * RSUB, RSUB)],
                    den_out.at[c, pl.ds(s * RSUB, RSUB)])

    @pl.when(s == NS - 1)
    def _():
        pltpu.sync_copy(acc_sp.at[pl.ds(NS * RSUB, RTAIL)],
                        acc_out.at[c, pl.ds(NS * RSUB, RTAIL)])
        pltpu.sync_copy(den_sp.at[pl.ds(NS * RSUB, RTAIL)],
                        den_out.at[c, pl.ds(NS * RSUB, RTAIL)])


def _sc_edge(hpads, a_s, a_d, src, dst, zeros_blk, zd):
    mesh = plsc.VectorSubcoreMesh(core_axis_name="c", subcore_axis_name="s",
                                  num_cores=NC, num_subcores=NS)
    f = pl.kernel(
        _sc_body,
        out_type=[
            jax.ShapeDtypeStruct((NC, N, DH), jnp.float32),
            jax.ShapeDtypeStruct((NC, N, DW), jnp.float32),
        ],
        mesh=mesh,
        compiler_params=pltpu.CompilerParams(needs_layout_passes=False,
                                             use_tc_tiling_on_sc=False),
        scratch_types=[
            pltpu.VMEM((NCHUNK, K), jnp.int32),
            pltpu.VMEM((NCHUNK, K), jnp.int32),
            pltpu.VMEM((N,), jnp.float32),
            pltpu.VMEM((N,), jnp.float32),
            pltpu.VMEM((NBW * K, DW), jnp.float32),
            pltpu.VMEM((NBG, K, DH), jnp.bfloat16),
            pltpu.VMEM((NBS, K, DH), jnp.float32),
            pltpu.VMEM_SHARED((N, DH), jnp.float32),
            pltpu.VMEM_SHARED((N, DW), jnp.float32),
            pltpu.SemaphoreType.DMA,
            pltpu.SemaphoreType.DMA,
            pltpu.SemaphoreType.DMA,
            pltpu.SemaphoreType.DMA,
        ],
    )
    return f(hpads, a_s, a_d, src, dst, zeros_blk, zd)


def kernel(x, edge_index, W1, att_src1, att_dst1, b1, W2, att_src2, att_dst2,
           b2):
    src = edge_index[0].reshape(NS, NCHUNK, K)
    dst = edge_index[1].reshape(NS, NCHUNK, K)
    zeros_blk = jnp.zeros((RSUB, DH), jnp.float32)
    zd = jnp.zeros((RSUB, DW), jnp.float32)
    pmat = jnp.eye(D, dtype=jnp.float32)[PERM]

    hpads1, a_s1, a_d1, sw1 = _tc_prep(
        x, W1[:, PERM], att_src1.reshape(D, 1)[PERM],
        att_dst1.reshape(D, 1)[PERM])
    acc1, den1 = _sc_edge(hpads1, a_s1.reshape(N), a_d1.reshape(N), src, dst,
                          zeros_blk, zd)
    hpads2, a_s2, a_d2, sw2 = _tc_mid(
        acc1, den1, sw1, hpads1, pmat, b1.reshape(1, D), W2[:, PERM],
        att_src2.reshape(D, 1)[PERM], att_dst2.reshape(D, 1)[PERM])
    acc2, den2 = _sc_edge(hpads2, a_s2.reshape(N), a_d2.reshape(N), src, dst,
                          zeros_blk, zd)
    return _tc_fin(acc2, den2, sw2, hpads2, pmat, b2.reshape(1, D))


# speedup vs baseline: 1.1652x; 1.1652x over previous
import numpy as np

import jax
import jax.numpy as jnp
from jax import lax
from jax.experimental import pallas as pl
from jax.experimental.pallas import tpu as pltpu
from jax.experimental.pallas import tpu_sc as plsc

N = 10000
E = 320000
D = 128
DH = 64
DW = 8
NC = 2
NS = 16
EW = E // NS
K = 80
NCHUNK = EW // K
NBG = 3
NBS = 3
NBW = 4
RSUB = 624
RTAIL = N - NS * RSUB
R = 1000

_g = np.arange(D) // 32 * 32
_t = np.arange(D) % 32
PERM = (_g + np.where(_t % 2 == 0, _t // 2, 16 + _t // 2)).astype(np.int32)


def _emit_prep(h, hpads_ref, as_ref, ad_ref, sw_ref, atts_ref, attd_ref):
    a_s = jnp.dot(h, atts_ref[...], preferred_element_type=jnp.float32)
    a_d = jnp.dot(h, attd_ref[...], preferred_element_type=jnp.float32)
    hpads_ref[0] = h[:, :DH].astype(jnp.bfloat16)
    hpads_ref[1] = h[:, DH:].astype(jnp.bfloat16)
    as_ref[...] = a_s
    ad_ref[...] = a_d
    s = a_s + a_d
    sw_ref[...] = jnp.exp(jnp.maximum(s, 0.2 * s))


def _combine(acc_ref, den_ref, sw_ref, hpads_ref, pmat_ref, b_ref):
    ht = jnp.concatenate([hpads_ref[0], hpads_ref[1]],
                         axis=1).astype(jnp.float32)
    h = jnp.dot(ht, pmat_ref[...], preferred_element_type=jnp.float32)
    sw = sw_ref[...]
    num = jnp.concatenate([acc_ref[0], acc_ref[1]], axis=1) + sw * h
    den = den_ref[0, :, 0:1] + sw
    return num / den + b_ref[...]


def _prep_body(z_ref, w_ref, atts_ref, attd_ref, hpads_ref, as_ref, ad_ref,
               sw_ref):
    h = jnp.dot(z_ref[...], w_ref[...], preferred_element_type=jnp.float32)
    _emit_prep(h, hpads_ref, as_ref, ad_ref, sw_ref, atts_ref, attd_ref)


def _mid_body(acc_ref, den_ref, sw_ref, hpads_ref, pmat_ref, b_ref, w_ref,
              atts_ref, attd_ref, hpads2_ref, as_ref, ad_ref, sw2_ref):
    z = jnp.maximum(_combine(acc_ref, den_ref, sw_ref, hpads_ref, pmat_ref,
                             b_ref), 0.0)
    h2 = jnp.dot(z, w_ref[...], preferred_element_type=jnp.float32)
    _emit_prep(h2, hpads2_ref, as_ref, ad_ref, sw2_ref, atts_ref, attd_ref)


def _fin_body(acc_ref, den_ref, sw_ref, hpads_ref, pmat_ref, b_ref, out_ref):
    out_ref[...] = _combine(acc_ref, den_ref, sw_ref, hpads_ref, pmat_ref,
                            b_ref)


_prep_outs = dict(
    out_specs=[
        pl.BlockSpec((NC, R, DH), lambda i: (0, i, 0)),
        pl.BlockSpec((R, 1), lambda i: (i, 0)),
        pl.BlockSpec((R, 1), lambda i: (i, 0)),
        pl.BlockSpec((R, 1), lambda i: (i, 0)),
    ],
    out_shape=[
        jax.ShapeDtypeStruct((NC, N, DH), jnp.bfloat16),
        jax.ShapeDtypeStruct((N, 1), jnp.float32),
        jax.ShapeDtypeStruct((N, 1), jnp.float32),
        jax.ShapeDtypeStruct((N, 1), jnp.float32),
    ],
)

_acc_specs = [
    pl.BlockSpec((NC, R, DH), lambda i: (0, i, 0)),
    pl.BlockSpec((NC, R, DW), lambda i: (0, i, 0)),
    pl.BlockSpec((R, 1), lambda i: (i, 0)),
    pl.BlockSpec((NC, R, DH), lambda i: (0, i, 0)),
    pl.BlockSpec((D, D), lambda i: (0, 0)),
    pl.BlockSpec((1, D), lambda i: (0, 0)),
]


def _tc_prep(z, W, att_s, att_d):
    return pl.pallas_call(
        _prep_body,
        grid=(N // R,),
        in_specs=[
            pl.BlockSpec((R, D), lambda i: (i, 0)),
            pl.BlockSpec((D, D), lambda i: (0, 0)),
            pl.BlockSpec((D, 1), lambda i: (0, 0)),
            pl.BlockSpec((D, 1), lambda i: (0, 0)),
        ],
        **_prep_outs,
    )(z, W, att_s, att_d)


def _tc_mid(acc, den, sw, hpads, pmat, b, W, att_s, att_d):
    return pl.pallas_call(
        _mid_body,
        grid=(N // R,),
        in_specs=_acc_specs + [
            pl.BlockSpec((D, D), lambda i: (0, 0)),
            pl.BlockSpec((D, 1), lambda i: (0, 0)),
            pl.BlockSpec((D, 1), lambda i: (0, 0)),
        ],
        **_prep_outs,
    )(acc, den, sw, hpads, pmat, b, W, att_s, att_d)


def _tc_fin(acc, den, sw, hpads, pmat, b):
    return pl.pallas_call(
        _fin_body,
        grid=(N // R,),
        in_specs=_acc_specs,
        out_specs=pl.BlockSpec((R, D), lambda i: (i, 0)),
        out_shape=jax.ShapeDtypeStruct((N, D), jnp.float32),
    )(acc, den, sw, hpads, pmat, b)


def _sc_body(hpads_hbm, as_hbm, ad_hbm, src_hbm, dst_hbm, zeros_hbm, zd_hbm,
             acc_out, den_out,
             srcv, dstv, astab, adtab, wbufs, rows_bf, rows_f, acc_sp, den_sp,
             gsem, ssem, wsem, psem):
    c = lax.axis_index("c")
    s = lax.axis_index("s")

    pltpu.async_copy(src_hbm.at[s], srcv, psem)
    pltpu.async_copy(dst_hbm.at[s], dstv, psem)
    pltpu.async_copy(as_hbm, astab, psem)
    pltpu.async_copy(ad_hbm, adtab, psem)
    pltpu.async_copy(zeros_hbm, acc_sp.at[pl.ds(s * RSUB, RSUB)], psem)
    pltpu.async_copy(zd_hbm.at[pl.ds(0, RSUB)],
                     den_sp.at[pl.ds(s * RSUB, RSUB)], psem)
    pltpu.async_copy(zd_hbm.at[pl.ds(0, NBW * K)], wbufs, psem)

    @pl.when(s == NS - 1)
    def _():
        pltpu.async_copy(zeros_hbm.at[pl.ds(0, RTAIL)],
                         acc_sp.at[pl.ds(NS * RSUB, RTAIL)], psem)
        pltpu.async_copy(zd_hbm.at[pl.ds(0, RTAIL)],
                         den_sp.at[pl.ds(NS * RSUB, RTAIL)], psem)

    pltpu.make_async_copy(src_hbm.at[s], srcv, psem).wait()
    pltpu.make_async_copy(dst_hbm.at[s], dstv, psem).wait()
    pltpu.make_async_copy(as_hbm, astab, psem).wait()
    pltpu.make_async_copy(ad_hbm, adtab, psem).wait()
    pltpu.make_async_copy(zeros_hbm, acc_sp.at[pl.ds(s * RSUB, RSUB)],
                          psem).wait()
    pltpu.make_async_copy(zd_hbm.at[pl.ds(0, RSUB)],
                          den_sp.at[pl.ds(s * RSUB, RSUB)], psem).wait()
    pltpu.make_async_copy(zd_hbm.at[pl.ds(0, NBW * K)], wbufs, psem).wait()

    @pl.when(s == NS - 1)
    def _():
        pltpu.make_async_copy(zeros_hbm.at[pl.ds(0, RTAIL)],
                              acc_sp.at[pl.ds(NS * RSUB, RTAIL)], psem).wait()
        pltpu.make_async_copy(zd_hbm.at[pl.ds(0, RTAIL)],
                              den_sp.at[pl.ds(NS * RSUB, RTAIL)], psem).wait()

    plsc.subcore_barrier()
    htab = hpads_hbm.at[c]
    col0 = jnp.zeros((16,), jnp.int32)
    lane = lax.iota(jnp.int32, 16)

    pltpu.async_copy(htab.at[srcv.at[0]], rows_bf.at[0], gsem)
    pltpu.async_copy(htab.at[srcv.at[1]], rows_bf.at[1], gsem)

    def chunk(i, carry):
        bg = lax.rem(i, NBG)
        bs = lax.rem(i, NBS)
        bw = lax.rem(i, NBW)

        @pl.when(i >= NBW)
        def _():
            pltpu.make_async_copy(
                wbufs.at[pl.ds(bw * K, K)],
                den_sp.at[dstv.at[i - NBW]], wsem).wait()

        for j in range(K // 16):
            sidx = srcv[i, pl.ds(j * 16, 16)]
            didx = dstv[i, pl.ds(j * 16, 16)]
            a = plsc.load_gather(astab, [sidx]) + plsc.load_gather(adtab, [didx])
            w16 = jnp.exp(jnp.maximum(a, 0.2 * a))
            plsc.store_scatter(wbufs, [bw * K + j * 16 + lane, col0], w16)
        pltpu.async_copy(wbufs.at[pl.ds(bw * K, K)],
                         den_sp.at[dstv.at[i]], wsem, add=True)

        @pl.when(i < NCHUNK - 2)
        def _():
            pltpu.async_copy(htab.at[srcv.at[i + 2]],
                             rows_bf.at[lax.rem(i + 2, NBG)], gsem)

        @pl.when(i >= NBS)
        def _():
            pltpu.make_async_copy(rows_f.at[bs],
                                  acc_sp.at[dstv.at[i - NBS]], ssem).wait()

        pltpu.make_async_copy(htab.at[srcv.at[i]], rows_bf.at[bg], gsem).wait()

        @plsc.parallel_loop(0, K, unroll=8)
        def scale(k):
            wv = plsc.load_gather(wbufs, [jnp.full((16,), bw * K + k,
                                                   jnp.int32), col0])
            for g in range(DH // 32):
                v = rows_bf[bg, k, pl.ds(g * 32, 32)]
                ev, od = plsc.unpack(v, format=plsc.PackFormat.INTERLEAVED,
                                     preferred_element_type=jnp.float32)
                rows_f[bs, k, pl.ds(g * 32, 16)] = ev * wv
                rows_f[bs, k, pl.ds(g * 32 + 16, 16)] = od * wv

        pltpu.async_copy(rows_f.at[bs], acc_sp.at[dstv.at[i]], ssem, add=True)
        return carry

    lax.fori_loop(0, NCHUNK, chunk, 0)
    for t in range(NBS):
        i = NCHUNK - NBS + t
        pltpu.make_async_copy(rows_f.at[lax.rem(i, NBS)],
                              acc_sp.at[dstv.at[i]], ssem).wait()
    for t in range(NBW):
        i = NCHUNK - NBW + t
        pltpu.make_async_copy(wbufs.at[pl.ds(lax.rem(i, NBW) * K, K)],
                              den_sp.at[dstv.at[i]], wsem).wait()
    plsc.subcore_barrier()
    pltpu.sync_copy(acc_sp.at[pl.ds(s * RSUB, RSUB)],
                    acc_out.at[c, pl.ds(s * RSUB, RSUB)])
    pltpu.sync_copy(den_sp.at[pl.ds(s * RSUB, RSUB)],
                    den_out.at[c, pl.ds(s * RSUB, RSUB)])

    @pl.when(s == NS - 1)
    def _():
        pltpu.sync_copy(acc_sp.at[pl.ds(NS * RSUB, RTAIL)],
                        acc_out.at[c, pl.ds(NS * RSUB, RTAIL)])
        pltpu.sync_copy(den_sp.at[pl.ds(NS * RSUB, RTAIL)],
                        den_out.at[c, pl.ds(NS * RSUB, RTAIL)])


def _sc_edge(hpads, a_s, a_d, src, dst, zeros_blk, zd):
    mesh = plsc.VectorSubcoreMesh(core_axis_name="c", subcore_axis_name="s",
                                  num_cores=NC, num_subcores=NS)
    f = pl.kernel(
        _sc_body,
        out_type=[
            jax.ShapeDtypeStruct((NC, N, DH), jnp.float32),
            jax.ShapeDtypeStruct((NC, N, DW), jnp.float32),
        ],
        mesh=mesh,
        compiler_params=pltpu.CompilerParams(needs_layout_passes=False,
                                             use_tc_tiling_on_sc=False),
        scratch_types=[
            pltpu.VMEM((NCHUNK, K), jnp.int32),
            pltpu.VMEM((NCHUNK, K), jnp.int32),
            pltpu.VMEM((N,), jnp.float32),
            pltpu.VMEM((N,), jnp.float32),
            pltpu.VMEM((NBW * K, DW), jnp.float32),
            pltpu.VMEM((NBG, K, DH), jnp.bfloat16),
            pltpu.VMEM((NBS, K, DH), jnp.float32),
            pltpu.VMEM_SHARED((N, DH), jnp.float32),
            pltpu.VMEM_SHARED((N, DW), jnp.float32),
            pltpu.SemaphoreType.DMA,
            pltpu.SemaphoreType.DMA,
            pltpu.SemaphoreType.DMA,
            pltpu.SemaphoreType.DMA,
        ],
    )
    return f(hpads, a_s, a_d, src, dst, zeros_blk, zd)


def kernel(x, edge_index, W1, att_src1, att_dst1, b1, W2, att_src2, att_dst2,
           b2):
    src = edge_index[0].reshape(NS, NCHUNK, K)
    dst = edge_index[1].reshape(NS, NCHUNK, K)
    zeros_blk = jnp.zeros((RSUB, DH), jnp.float32)
    zd = jnp.zeros((RSUB, DW), jnp.float32)
    pmat = jnp.eye(D, dtype=jnp.float32)[PERM]

    hpads1, a_s1, a_d1, sw1 = _tc_prep(
        x, W1[:, PERM], att_src1.reshape(D, 1)[PERM],
        att_dst1.reshape(D, 1)[PERM])
    acc1, den1 = _sc_edge(hpads1, a_s1.reshape(N), a_d1.reshape(N), src, dst,
                          zeros_blk, zd)
    hpads2, a_s2, a_d2, sw2 = _tc_mid(
        acc1, den1, sw1, hpads1, pmat, b1.reshape(1, D), W2[:, PERM],
        att_src2.reshape(D, 1)[PERM], att_dst2.reshape(D, 1)[PERM])
    acc2, den2 = _sc_edge(hpads2, a_s2.reshape(N), a_d2.reshape(N), src, dst,
                          zeros_blk, zd)
    return _tc_fin(acc2, den2, sw2, hpads2, pmat, b2.reshape(1, D))


# speedup vs baseline: 1.1773x; 1.0104x over previous
import numpy as np

import jax
import jax.numpy as jnp
from jax import lax
from jax.experimental import pallas as pl
from jax.experimental.pallas import tpu as pltpu
from jax.experimental.pallas import tpu_sc as plsc

N = 10000
E = 320000
D = 128
DH = 64
DW = 8
NC = 2
NS = 16
EW = E // NS
K = 80
NCHUNK = EW // K
NBG = 3
NBS = 3
NBW = 4
RSUB = 624
RTAIL = N - NS * RSUB
R = 1000

_g = np.arange(D) // 32 * 32
_t = np.arange(D) % 32
PERM = (_g + np.where(_t % 2 == 0, _t // 2, 16 + _t // 2)).astype(np.int32)


def _emit_prep(h, hpads_ref, as_ref, ad_ref, sw_ref, atts_ref, attd_ref):
    a_s = jnp.dot(h, atts_ref[...], preferred_element_type=jnp.float32)
    a_d = jnp.dot(h, attd_ref[...], preferred_element_type=jnp.float32)
    hpads_ref[0] = h[:, :DH].astype(jnp.bfloat16)
    hpads_ref[1] = h[:, DH:].astype(jnp.bfloat16)
    as_ref[...] = a_s
    ad_ref[...] = a_d
    s = a_s + a_d
    sw_ref[...] = jnp.exp(jnp.maximum(s, 0.2 * s))


def _combine(acc_ref, den_ref, sw_ref, hpads_ref, pmat_ref, b_ref):
    ht = jnp.concatenate([hpads_ref[0], hpads_ref[1]],
                         axis=1).astype(jnp.float32)
    h = jnp.dot(ht, pmat_ref[...], preferred_element_type=jnp.float32)
    sw = sw_ref[...]
    num = jnp.concatenate([acc_ref[0], acc_ref[1]], axis=1) + sw * h
    den = den_ref[0, :, 0:1] + sw
    return num / den + b_ref[...]


def _prep_body(z_ref, w_ref, atts_ref, attd_ref, hpads_ref, as_ref, ad_ref,
               sw_ref):
    h = jnp.dot(z_ref[...], w_ref[...], preferred_element_type=jnp.float32)
    _emit_prep(h, hpads_ref, as_ref, ad_ref, sw_ref, atts_ref, attd_ref)


def _mid_body(acc_ref, den_ref, sw_ref, hpads_ref, pmat_ref, b_ref, w_ref,
              atts_ref, attd_ref, hpads2_ref, as_ref, ad_ref, sw2_ref):
    z = jnp.maximum(_combine(acc_ref, den_ref, sw_ref, hpads_ref, pmat_ref,
                             b_ref), 0.0)
    h2 = jnp.dot(z, w_ref[...], preferred_element_type=jnp.float32)
    _emit_prep(h2, hpads2_ref, as_ref, ad_ref, sw2_ref, atts_ref, attd_ref)


def _fin_body(acc_ref, den_ref, sw_ref, hpads_ref, pmat_ref, b_ref, out_ref):
    out_ref[...] = _combine(acc_ref, den_ref, sw_ref, hpads_ref, pmat_ref,
                            b_ref)


_prep_outs = dict(
    out_specs=[
        pl.BlockSpec((NC, R, DH), lambda i: (0, i, 0)),
        pl.BlockSpec((R, 1), lambda i: (i, 0)),
        pl.BlockSpec((R, 1), lambda i: (i, 0)),
        pl.BlockSpec((R, 1), lambda i: (i, 0)),
    ],
    out_shape=[
        jax.ShapeDtypeStruct((NC, N, DH), jnp.bfloat16),
        jax.ShapeDtypeStruct((N, 1), jnp.float32),
        jax.ShapeDtypeStruct((N, 1), jnp.float32),
        jax.ShapeDtypeStruct((N, 1), jnp.float32),
    ],
)

_acc_specs = [
    pl.BlockSpec((NC, R, DH), lambda i: (0, i, 0)),
    pl.BlockSpec((NC, R, DW), lambda i: (0, i, 0)),
    pl.BlockSpec((R, 1), lambda i: (i, 0)),
    pl.BlockSpec((NC, R, DH), lambda i: (0, i, 0)),
    pl.BlockSpec((D, D), lambda i: (0, 0)),
    pl.BlockSpec((1, D), lambda i: (0, 0)),
]


def _tc_prep(z, W, att_s, att_d):
    return pl.pallas_call(
        _prep_body,
        grid=(N // R,),
        in_specs=[
            pl.BlockSpec((R, D), lambda i: (i, 0)),
            pl.BlockSpec((D, D), lambda i: (0, 0)),
            pl.BlockSpec((D, 1), lambda i: (0, 0)),
            pl.BlockSpec((D, 1), lambda i: (0, 0)),
        ],
        **_prep_outs,
    )(z, W, att_s, att_d)


def _tc_mid(acc, den, sw, hpads, pmat, b, W, att_s, att_d):
    return pl.pallas_call(
        _mid_body,
        grid=(N // R,),
        in_specs=_acc_specs + [
            pl.BlockSpec((D, D), lambda i: (0, 0)),
            pl.BlockSpec((D, 1), lambda i: (0, 0)),
            pl.BlockSpec((D, 1), lambda i: (0, 0)),
        ],
        **_prep_outs,
    )(acc, den, sw, hpads, pmat, b, W, att_s, att_d)


def _tc_fin(acc, den, sw, hpads, pmat, b):
    return pl.pallas_call(
        _fin_body,
        grid=(N // R,),
        in_specs=_acc_specs,
        out_specs=pl.BlockSpec((R, D), lambda i: (i, 0)),
        out_shape=jax.ShapeDtypeStruct((N, D), jnp.float32),
    )(acc, den, sw, hpads, pmat, b)


def _sc_body(hpads_hbm, as_hbm, ad_hbm, src_hbm, dst_hbm, zeros_hbm, zd_hbm,
             acc_out, den_out,
             srcv, dstv, astab, adtab, wbufs, rows_bf, rows_f, acc_sp, den_sp,
             gsem, ssem, wsem, psem):
    c = lax.axis_index("c")
    s = lax.axis_index("s")

    pltpu.async_copy(src_hbm.at[s], srcv, psem)
    pltpu.async_copy(dst_hbm.at[s], dstv, psem)
    pltpu.async_copy(as_hbm, astab, psem)
    pltpu.async_copy(ad_hbm, adtab, psem)
    pltpu.async_copy(zeros_hbm, acc_sp.at[pl.ds(s * RSUB, RSUB)], psem)
    pltpu.async_copy(zd_hbm.at[pl.ds(0, RSUB)],
                     den_sp.at[pl.ds(s * RSUB, RSUB)], psem)
    pltpu.async_copy(zd_hbm.at[pl.ds(0, NBW * K)], wbufs, psem)

    @pl.when(s == NS - 1)
    def _():
        pltpu.async_copy(zeros_hbm.at[pl.ds(0, RTAIL)],
                         acc_sp.at[pl.ds(NS * RSUB, RTAIL)], psem)
        pltpu.async_copy(zd_hbm.at[pl.ds(0, RTAIL)],
                         den_sp.at[pl.ds(NS * RSUB, RTAIL)], psem)

    pltpu.make_async_copy(src_hbm.at[s], srcv, psem).wait()
    htab = hpads_hbm.at[c]
    pltpu.async_copy(htab.at[srcv.at[0]], rows_bf.at[0], gsem)
    pltpu.async_copy(htab.at[srcv.at[1]], rows_bf.at[1], gsem)
    pltpu.make_async_copy(dst_hbm.at[s], dstv, psem).wait()
    pltpu.make_async_copy(as_hbm, astab, psem).wait()
    pltpu.make_async_copy(ad_hbm, adtab, psem).wait()
    pltpu.make_async_copy(zeros_hbm, acc_sp.at[pl.ds(s * RSUB, RSUB)],
                          psem).wait()
    pltpu.make_async_copy(zd_hbm.at[pl.ds(0, RSUB)],
                          den_sp.at[pl.ds(s * RSUB, RSUB)], psem).wait()
    pltpu.make_async_copy(zd_hbm.at[pl.ds(0, NBW * K)], wbufs, psem).wait()

    @pl.when(s == NS - 1)
    def _():
        pltpu.make_async_copy(zeros_hbm.at[pl.ds(0, RTAIL)],
                              acc_sp.at[pl.ds(NS * RSUB, RTAIL)], psem).wait()
        pltpu.make_async_copy(zd_hbm.at[pl.ds(0, RTAIL)],
                              den_sp.at[pl.ds(NS * RSUB, RTAIL)], psem).wait()

    plsc.subcore_barrier()
    col0 = jnp.zeros((16,), jnp.int32)
    lane = lax.iota(jnp.int32, 16)

    def chunk(i, carry):
        bg = lax.rem(i, NBG)
        bs = lax.rem(i, NBS)
        bw = lax.rem(i, NBW)

        @pl.when(i >= NBW)
        def _():
            pltpu.make_async_copy(
                wbufs.at[pl.ds(bw * K, K)],
                den_sp.at[dstv.at[i - NBW]], wsem).wait()

        for j in range(K // 16):
            sidx = srcv[i, pl.ds(j * 16, 16)]
            didx = dstv[i, pl.ds(j * 16, 16)]
            a = plsc.load_gather(astab, [sidx]) + plsc.load_gather(adtab, [didx])
            w16 = jnp.exp(jnp.maximum(a, 0.2 * a))
            plsc.store_scatter(wbufs, [bw * K + j * 16 + lane, col0], w16)
        pltpu.async_copy(wbufs.at[pl.ds(bw * K, K)],
                         den_sp.at[dstv.at[i]], wsem, add=True)

        @pl.when(i < NCHUNK - 2)
        def _():
            pltpu.async_copy(htab.at[srcv.at[i + 2]],
                             rows_bf.at[lax.rem(i + 2, NBG)], gsem)

        @pl.when(i >= NBS)
        def _():
            pltpu.make_async_copy(rows_f.at[bs],
                                  acc_sp.at[dstv.at[i - NBS]], ssem).wait()

        pltpu.make_async_copy(htab.at[srcv.at[i]], rows_bf.at[bg], gsem).wait()

        @plsc.parallel_loop(0, K, unroll=8)
        def scale(k):
            wv = plsc.load_gather(wbufs, [jnp.full((16,), bw * K + k,
                                                   jnp.int32), col0])
            for g in range(DH // 32):
                v = rows_bf[bg, k, pl.ds(g * 32, 32)]
                ev, od = plsc.unpack(v, format=plsc.PackFormat.INTERLEAVED,
                                     preferred_element_type=jnp.float32)
                rows_f[bs, k, pl.ds(g * 32, 16)] = ev * wv
                rows_f[bs, k, pl.ds(g * 32 + 16, 16)] = od * wv

        pltpu.async_copy(rows_f.at[bs], acc_sp.at[dstv.at[i]], ssem, add=True)
        return carry

    lax.fori_loop(0, NCHUNK, chunk, 0)
    for t in range(NBS):
        i = NCHUNK - NBS + t
        pltpu.make_async_copy(rows_f.at[lax.rem(i, NBS)],
                              acc_sp.at[dstv.at[i]], ssem).wait()
    for t in range(NBW):
        i = NCHUNK - NBW + t
        pltpu.make_async_copy(wbufs.at[pl.ds(lax.rem(i, NBW) * K, K)],
                              den_sp.at[dstv.at[i]], wsem).wait()
    plsc.subcore_barrier()
    pltpu.async_copy(acc_sp.at[pl.ds(s * RSUB, RSUB)],
                     acc_out.at[c, pl.ds(s * RSUB, RSUB)], psem)
    pltpu.async_copy(den_sp.at[pl.ds(s * RSUB, RSUB)],
                     den_out.at[c, pl.ds(s * RSUB, RSUB)], psem)

    @pl.when(s == NS - 1)
    def _():
        pltpu.async_copy(acc_sp.at[pl.ds(NS * RSUB, RTAIL)],
                         acc_out.at[c, pl.ds(NS * RSUB, RTAIL)], psem)
        pltpu.async_copy(den_sp.at[pl.ds(NS * RSUB, RTAIL)],
                         den_out.at[c, pl.ds(NS * RSUB, RTAIL)], psem)

    pltpu.make_async_copy(acc_sp.at[pl.ds(s * RSUB, RSUB)],
                          acc_out.at[c, pl.ds(s * RSUB, RSUB)], psem).wait()
    pltpu.make_async_copy(den_sp.at[pl.ds(s * RSUB, RSUB)],
                          den_out.at[c, pl.ds(s * RSUB, RSUB)], psem).wait()

    @pl.when(s == NS - 1)
    def _():
        pltpu.make_async_copy(acc_sp.at[pl.ds(NS * RSUB, RTAIL)],
                              acc_out.at[c, pl.ds(NS * RSUB, RTAIL)],
                              psem).wait()
        pltpu.make_async_copy(den_sp.at[pl.ds(NS * RSUB, RTAIL)],
                              den_out.at[c, pl.ds(NS * RSUB, RTAIL)],
                              psem).wait()


def _sc_edge(hpads, a_s, a_d, src, dst, zeros_blk, zd):
    mesh = plsc.VectorSubcoreMesh(core_axis_name="c", subcore_axis_name="s",
                                  num_cores=NC, num_subcores=NS)
    f = pl.kernel(
        _sc_body,
        out_type=[
            jax.ShapeDtypeStruct((NC, N, DH), jnp.float32),
            jax.ShapeDtypeStruct((NC, N, DW), jnp.float32),
        ],
        mesh=mesh,
        compiler_params=pltpu.CompilerParams(needs_layout_passes=False,
                                             use_tc_tiling_on_sc=False),
        scratch_types=[
            pltpu.VMEM((NCHUNK, K), jnp.int32),
            pltpu.VMEM((NCHUNK, K), jnp.int32),
            pltpu.VMEM((N,), jnp.float32),
            pltpu.VMEM((N,), jnp.float32),
            pltpu.VMEM((NBW * K, DW), jnp.float32),
            pltpu.VMEM((NBG, K, DH), jnp.bfloat16),
            pltpu.VMEM((NBS, K, DH), jnp.float32),
            pltpu.VMEM_SHARED((N, DH), jnp.float32),
            pltpu.VMEM_SHARED((N, DW), jnp.float32),
            pltpu.SemaphoreType.DMA,
            pltpu.SemaphoreType.DMA,
            pltpu.SemaphoreType.DMA,
            pltpu.SemaphoreType.DMA,
        ],
    )
    return f(hpads, a_s, a_d, src, dst, zeros_blk, zd)


def kernel(x, edge_index, W1, att_src1, att_dst1, b1, W2, att_src2, att_dst2,
           b2):
    src = edge_index[0].reshape(NS, NCHUNK, K)
    dst = edge_index[1].reshape(NS, NCHUNK, K)
    zeros_blk = jnp.zeros((RSUB, DH), jnp.float32)
    zd = jnp.zeros((RSUB, DW), jnp.float32)
    pmat = jnp.eye(D, dtype=jnp.float32)[PERM]

    hpads1, a_s1, a_d1, sw1 = _tc_prep(
        x, W1[:, PERM], att_src1.reshape(D, 1)[PERM],
        att_dst1.reshape(D, 1)[PERM])
    acc1, den1 = _sc_edge(hpads1, a_s1.reshape(N), a_d1.reshape(N), src, dst,
                          zeros_blk, zd)
    hpads2, a_s2, a_d2, sw2 = _tc_mid(
        acc1, den1, sw1, hpads1, pmat, b1.reshape(1, D), W2[:, PERM],
        att_src2.reshape(D, 1)[PERM], att_dst2.reshape(D, 1)[PERM])
    acc2, den2 = _sc_edge(hpads2, a_s2.reshape(N), a_d2.reshape(N), src, dst,
                          zeros_blk, zd)
    return _tc_fin(acc2, den2, sw2, hpads2, pmat, b2.reshape(1, D))


# speedup vs baseline: 1.1930x; 1.0133x over previous
import numpy as np

import jax
import jax.numpy as jnp
from jax import lax
from jax.experimental import pallas as pl
from jax.experimental.pallas import tpu as pltpu
from jax.experimental.pallas import tpu_sc as plsc

N = 10000
E = 320000
D = 128
DH = 64
DW = 8
NC = 2
NS = 16
EW = E // NS
K = 80
NCHUNK = EW // K
NBG = 3
NBS = 3
NBW = 4
RSUB = 624
RTAIL = N - NS * RSUB
R = 1000

_g = np.arange(D) // 32 * 32
_t = np.arange(D) % 32
PERM = (_g + np.where(_t % 2 == 0, _t // 2, 16 + _t // 2)).astype(np.int32)


def _emit_prep(h, hpads_ref, as_ref, ad_ref, sw_ref, atts_ref, attd_ref):
    a_s = jnp.dot(h, atts_ref[...], preferred_element_type=jnp.float32)
    a_d = jnp.dot(h, attd_ref[...], preferred_element_type=jnp.float32)
    hpads_ref[0] = h[:, :DH].astype(jnp.bfloat16)
    hpads_ref[1] = h[:, DH:].astype(jnp.bfloat16)
    as_ref[...] = a_s
    ad_ref[...] = a_d
    s = a_s + a_d
    sw_ref[...] = jnp.exp(jnp.maximum(s, 0.2 * s))


def _combine(acc_ref, den_ref, sw_ref, hpads_ref, pmat_ref, b_ref):
    ht = jnp.concatenate([hpads_ref[0], hpads_ref[1]],
                         axis=1).astype(jnp.float32)
    h = jnp.dot(ht, pmat_ref[...], preferred_element_type=jnp.float32)
    sw = sw_ref[...]
    num = jnp.concatenate([acc_ref[0], acc_ref[1]], axis=1) + sw * h
    den = den_ref[0, :, 0:1] + sw
    return num / den + b_ref[...]


def _prep_body(z_ref, w_ref, atts_ref, attd_ref, hpads_ref, as_ref, ad_ref,
               sw_ref):
    h = jnp.dot(z_ref[...], w_ref[...], preferred_element_type=jnp.float32)
    _emit_prep(h, hpads_ref, as_ref, ad_ref, sw_ref, atts_ref, attd_ref)


def _mid_body(acc_ref, den_ref, sw_ref, hpads_ref, pmat_ref, b_ref, w_ref,
              atts_ref, attd_ref, hpads2_ref, as_ref, ad_ref, sw2_ref):
    z = jnp.maximum(_combine(acc_ref, den_ref, sw_ref, hpads_ref, pmat_ref,
                             b_ref), 0.0)
    h2 = jnp.dot(z, w_ref[...], preferred_element_type=jnp.float32)
    _emit_prep(h2, hpads2_ref, as_ref, ad_ref, sw2_ref, atts_ref, attd_ref)


def _fin_body(acc_ref, den_ref, sw_ref, hpads_ref, pmat_ref, b_ref, out_ref):
    out_ref[...] = _combine(acc_ref, den_ref, sw_ref, hpads_ref, pmat_ref,
                            b_ref)


_prep_outs = dict(
    out_specs=[
        pl.BlockSpec((NC, R, DH), lambda i: (0, i, 0)),
        pl.BlockSpec((R, 1), lambda i: (i, 0)),
        pl.BlockSpec((R, 1), lambda i: (i, 0)),
        pl.BlockSpec((R, 1), lambda i: (i, 0)),
    ],
    out_shape=[
        jax.ShapeDtypeStruct((NC, N, DH), jnp.bfloat16),
        jax.ShapeDtypeStruct((N, 1), jnp.float32),
        jax.ShapeDtypeStruct((N, 1), jnp.float32),
        jax.ShapeDtypeStruct((N, 1), jnp.float32),
    ],
)

_acc_specs = [
    pl.BlockSpec((NC, R, DH), lambda i: (0, i, 0)),
    pl.BlockSpec((NC, R, DW), lambda i: (0, i, 0)),
    pl.BlockSpec((R, 1), lambda i: (i, 0)),
    pl.BlockSpec((NC, R, DH), lambda i: (0, i, 0)),
    pl.BlockSpec((D, D), lambda i: (0, 0)),
    pl.BlockSpec((1, D), lambda i: (0, 0)),
]


def _tc_prep(z, W, att_s, att_d):
    return pl.pallas_call(
        _prep_body,
        grid=(N // R,),
        in_specs=[
            pl.BlockSpec((R, D), lambda i: (i, 0)),
            pl.BlockSpec((D, D), lambda i: (0, 0)),
            pl.BlockSpec((D, 1), lambda i: (0, 0)),
            pl.BlockSpec((D, 1), lambda i: (0, 0)),
        ],
        **_prep_outs,
    )(z, W, att_s, att_d)


def _tc_mid(acc, den, sw, hpads, pmat, b, W, att_s, att_d):
    return pl.pallas_call(
        _mid_body,
        grid=(N // R,),
        in_specs=_acc_specs + [
            pl.BlockSpec((D, D), lambda i: (0, 0)),
            pl.BlockSpec((D, 1), lambda i: (0, 0)),
            pl.BlockSpec((D, 1), lambda i: (0, 0)),
        ],
        **_prep_outs,
    )(acc, den, sw, hpads, pmat, b, W, att_s, att_d)


def _tc_fin(acc, den, sw, hpads, pmat, b):
    return pl.pallas_call(
        _fin_body,
        grid=(N // R,),
        in_specs=_acc_specs,
        out_specs=pl.BlockSpec((R, D), lambda i: (i, 0)),
        out_shape=jax.ShapeDtypeStruct((N, D), jnp.float32),
    )(acc, den, sw, hpads, pmat, b)


def _sc_body(hpads_hbm, as_hbm, ad_hbm, src_hbm, dst_hbm, zeros_hbm, zd_hbm,
             acc_out, den_out,
             srcv, dstv, astab, adtab, wbufs, rows_bf, rows_f, acc_sp, den_sp,
             gsem, ssem, wsem, psem):
    c = lax.axis_index("c")
    s = lax.axis_index("s")

    pltpu.async_copy(src_hbm.at[s], srcv, psem)
    pltpu.async_copy(dst_hbm.at[s], dstv, psem)
    pltpu.async_copy(as_hbm, astab, psem)
    pltpu.async_copy(ad_hbm, adtab, psem)
    pltpu.async_copy(zeros_hbm, acc_sp.at[pl.ds(s * RSUB, RSUB)], psem)
    pltpu.async_copy(zd_hbm.at[pl.ds(0, RSUB)],
                     den_sp.at[pl.ds(s * RSUB, RSUB)], psem)
    pltpu.async_copy(zd_hbm.at[pl.ds(0, NBW * K)], wbufs, psem)

    @pl.when(s == NS - 1)
    def _():
        pltpu.async_copy(zeros_hbm.at[pl.ds(0, RTAIL)],
                         acc_sp.at[pl.ds(NS * RSUB, RTAIL)], psem)
        pltpu.async_copy(zd_hbm.at[pl.ds(0, RTAIL)],
                         den_sp.at[pl.ds(NS * RSUB, RTAIL)], psem)

    pltpu.make_async_copy(src_hbm.at[s], srcv, psem).wait()
    htab = hpads_hbm.at[c]
    pltpu.async_copy(htab.at[srcv.at[0]], rows_bf.at[0], gsem)
    pltpu.async_copy(htab.at[srcv.at[1]], rows_bf.at[1], gsem)
    pltpu.make_async_copy(dst_hbm.at[s], dstv, psem).wait()
    pltpu.make_async_copy(as_hbm, astab, psem).wait()
    pltpu.make_async_copy(ad_hbm, adtab, psem).wait()
    pltpu.make_async_copy(zeros_hbm, acc_sp.at[pl.ds(s * RSUB, RSUB)],
                          psem).wait()
    pltpu.make_async_copy(zd_hbm.at[pl.ds(0, RSUB)],
                          den_sp.at[pl.ds(s * RSUB, RSUB)], psem).wait()
    pltpu.make_async_copy(zd_hbm.at[pl.ds(0, NBW * K)], wbufs, psem).wait()

    @pl.when(s == NS - 1)
    def _():
        pltpu.make_async_copy(zeros_hbm.at[pl.ds(0, RTAIL)],
                              acc_sp.at[pl.ds(NS * RSUB, RTAIL)], psem).wait()
        pltpu.make_async_copy(zd_hbm.at[pl.ds(0, RTAIL)],
                              den_sp.at[pl.ds(NS * RSUB, RTAIL)], psem).wait()

    plsc.subcore_barrier()
    col0 = jnp.zeros((16,), jnp.int32)
    lane = lax.iota(jnp.int32, 16)

    def chunk(i, carry):
        bg = lax.rem(i, NBG)
        bs = lax.rem(i, NBS)
        bw = lax.rem(i, NBW)

        @pl.when(i >= NBW)
        def _():
            pltpu.make_async_copy(
                wbufs.at[pl.ds(bw * K, K)],
                den_sp.at[dstv.at[i - NBW]], wsem).wait()

        @pl.when(i < NCHUNK - 2)
        def _():
            pltpu.async_copy(htab.at[srcv.at[i + 2]],
                             rows_bf.at[lax.rem(i + 2, NBG)], gsem)

        for j in range(K // 16):
            sidx = srcv[i, pl.ds(j * 16, 16)]
            didx = dstv[i, pl.ds(j * 16, 16)]
            a = plsc.load_gather(astab, [sidx]) + plsc.load_gather(adtab, [didx])
            w16 = jnp.exp(jnp.maximum(a, 0.2 * a))
            plsc.store_scatter(wbufs, [bw * K + j * 16 + lane, col0], w16)
        pltpu.async_copy(wbufs.at[pl.ds(bw * K, K)],
                         den_sp.at[dstv.at[i]], wsem, add=True)

        @pl.when(i >= NBS)
        def _():
            pltpu.make_async_copy(rows_f.at[bs],
                                  acc_sp.at[dstv.at[i - NBS]], ssem).wait()

        pltpu.make_async_copy(htab.at[srcv.at[i]], rows_bf.at[bg], gsem).wait()

        @plsc.parallel_loop(0, K, unroll=8)
        def scale(k):
            wv = plsc.load_gather(wbufs, [jnp.full((16,), bw * K + k,
                                                   jnp.int32), col0])
            for g in range(DH // 32):
                v = rows_bf[bg, k, pl.ds(g * 32, 32)]
                ev, od = plsc.unpack(v, format=plsc.PackFormat.INTERLEAVED,
                                     preferred_element_type=jnp.float32)
                rows_f[bs, k, pl.ds(g * 32, 16)] = ev * wv
                rows_f[bs, k, pl.ds(g * 32 + 16, 16)] = od * wv

        pltpu.async_copy(rows_f.at[bs], acc_sp.at[dstv.at[i]], ssem, add=True)
        return carry

    lax.fori_loop(0, NCHUNK, chunk, 0)
    for t in range(NBS):
        i = NCHUNK - NBS + t
        pltpu.make_async_copy(rows_f.at[lax.rem(i, NBS)],
                              acc_sp.at[dstv.at[i]], ssem).wait()
    for t in range(NBW):
        i = NCHUNK - NBW + t
        pltpu.make_async_copy(wbufs.at[pl.ds(lax.rem(i, NBW) * K, K)],
                              den_sp.at[dstv.at[i]], wsem).wait()
    plsc.subcore_barrier()
    pltpu.async_copy(acc_sp.at[pl.ds(s * RSUB, RSUB)],
                     acc_out.at[c, pl.ds(s * RSUB, RSUB)], psem)
    pltpu.async_copy(den_sp.at[pl.ds(s * RSUB, RSUB)],
                     den_out.at[c, pl.ds(s * RSUB, RSUB)], psem)

    @pl.when(s == NS - 1)
    def _():
        pltpu.async_copy(acc_sp.at[pl.ds(NS * RSUB, RTAIL)],
                         acc_out.at[c, pl.ds(NS * RSUB, RTAIL)], psem)
        pltpu.async_copy(den_sp.at[pl.ds(NS * RSUB, RTAIL)],
                         den_out.at[c, pl.ds(NS * RSUB, RTAIL)], psem)

    pltpu.make_async_copy(acc_sp.at[pl.ds(s * RSUB, RSUB)],
                          acc_out.at[c, pl.ds(s * RSUB, RSUB)], psem).wait()
    pltpu.make_async_copy(den_sp.at[pl.ds(s * RSUB, RSUB)],
                          den_out.at[c, pl.ds(s * RSUB, RSUB)], psem).wait()

    @pl.when(s == NS - 1)
    def _():
        pltpu.make_async_copy(acc_sp.at[pl.ds(NS * RSUB, RTAIL)],
                              acc_out.at[c, pl.ds(NS * RSUB, RTAIL)],
                              psem).wait()
        pltpu.make_async_copy(den_sp.at[pl.ds(NS * RSUB, RTAIL)],
                              den_out.at[c, pl.ds(NS * RSUB, RTAIL)],
                              psem).wait()


def _sc_edge(hpads, a_s, a_d, src, dst, zeros_blk, zd):
    mesh = plsc.VectorSubcoreMesh(core_axis_name="c", subcore_axis_name="s",
                                  num_cores=NC, num_subcores=NS)
    f = pl.kernel(
        _sc_body,
        out_type=[
            jax.ShapeDtypeStruct((NC, N, DH), jnp.float32),
            jax.ShapeDtypeStruct((NC, N, DW), jnp.float32),
        ],
        mesh=mesh,
        compiler_params=pltpu.CompilerParams(needs_layout_passes=False,
                                             use_tc_tiling_on_sc=False),
        scratch_types=[
            pltpu.VMEM((NCHUNK, K), jnp.int32),
            pltpu.VMEM((NCHUNK, K), jnp.int32),
            pltpu.VMEM((N,), jnp.float32),
            pltpu.VMEM((N,), jnp.float32),
            pltpu.VMEM((NBW * K, DW), jnp.float32),
            pltpu.VMEM((NBG, K, DH), jnp.bfloat16),
            pltpu.VMEM((NBS, K, DH), jnp.float32),
            pltpu.VMEM_SHARED((N, DH), jnp.float32),
            pltpu.VMEM_SHARED((N, DW), jnp.float32),
            pltpu.SemaphoreType.DMA,
            pltpu.SemaphoreType.DMA,
            pltpu.SemaphoreType.DMA,
            pltpu.SemaphoreType.DMA,
        ],
    )
    return f(hpads, a_s, a_d, src, dst, zeros_blk, zd)


def kernel(x, edge_index, W1, att_src1, att_dst1, b1, W2, att_src2, att_dst2,
           b2):
    src = edge_index[0].reshape(NS, NCHUNK, K)
    dst = edge_index[1].reshape(NS, NCHUNK, K)
    zeros_blk = jnp.zeros((RSUB, DH), jnp.float32)
    zd = jnp.zeros((RSUB, DW), jnp.float32)
    pmat = jnp.eye(D, dtype=jnp.float32)[PERM]

    hpads1, a_s1, a_d1, sw1 = _tc_prep(
        x, W1[:, PERM], att_src1.reshape(D, 1)[PERM],
        att_dst1.reshape(D, 1)[PERM])
    acc1, den1 = _sc_edge(hpads1, a_s1.reshape(N), a_d1.reshape(N), src, dst,
                          zeros_blk, zd)
    hpads2, a_s2, a_d2, sw2 = _tc_mid(
        acc1, den1, sw1, hpads1, pmat, b1.reshape(1, D), W2[:, PERM],
        att_src2.reshape(D, 1)[PERM], att_dst2.reshape(D, 1)[PERM])
    acc2, den2 = _sc_edge(hpads2, a_s2.reshape(N), a_d2.reshape(N), src, dst,
                          zeros_blk, zd)
    return _tc_fin(acc2, den2, sw2, hpads2, pmat, b2.reshape(1, D))
